# Initial kernel scaffold; baseline (speedup 1.0000x reference)
import jax
import jax.numpy as jnp
from jax.experimental import pallas as pl


def kernel(x, edge_index, edge_weights, W_emb, b_emb, W_gcn, b_gcn, W_dec, b_dec):
    raise NotImplementedError("write your pallas kernel here")


# speedup vs baseline: 5.2440x; 5.2440x over previous
import functools

import jax
import jax.numpy as jnp
from jax import lax
from jax.experimental import pallas as pl
from jax.experimental.pallas import tpu as pltpu
from jax.experimental.pallas import tpu_sc as plsc

N = 10000
E = 320000
DIN = 192
H = 128
DOUT = 96
NLAYERS = 5

NC = 2
NS = 16
B = 128
EPT = 10112
NBLK = EPT // B
EPAD = NC * NS * EPT
NPAD = 10240
RPT = NPAD // NS
RCHUNK = 128
NCHUNK = RPT // RCHUNK

_MESH = dict(core_axis_name="c", subcore_axis_name="s", num_cores=NC,
             num_subcores=NS)


def _zero_rows(buf, nrows, width):
    def body(i, carry):
        for j in range(width // 16):
            buf[i, pl.ds(j * 16, 16)] = jnp.zeros((16,), jnp.float32)
        return carry
    lax.fori_loop(0, nrows, body, 0)


def _msg_scatter(ts, rowp, colp, wp):
    mesh = plsc.VectorSubcoreMesh(**_MESH)

    @functools.partial(
        pl.kernel,
        out_type=jax.ShapeDtypeStruct((NC, NPAD, H), jnp.float32),
        mesh=mesh,
        scratch_types=[
            pltpu.VMEM((B,), jnp.int32),
            pltpu.VMEM((B,), jnp.int32),
            pltpu.VMEM((B,), jnp.float32),
            pltpu.VMEM((B, H), jnp.float32),
            pltpu.VMEM_SHARED((NPAD, H), jnp.float32),
            pltpu.SemaphoreType.DMA,
        ],
    )
    def k(ts_hbm, row_hbm, col_hbm, w_hbm, out_hbm,
          row_v, col_v, w_v, msg_v, acc_sh, sem):
        cid = lax.axis_index("c")
        sid = lax.axis_index("s")
        _zero_rows(msg_v, B, H)
        for kk in range(NCHUNK):
            off = sid * RPT + kk * RCHUNK
            pltpu.sync_copy(msg_v, acc_sh.at[pl.ds(off, RCHUNK)])
        plsc.subcore_barrier()
        ebase = cid * (EPAD // NC) + sid * EPT

        def blk(g, carry):
            base = ebase + g * B
            pltpu.sync_copy(row_hbm.at[pl.ds(base, B)], row_v)
            pltpu.sync_copy(col_hbm.at[pl.ds(base, B)], col_v)
            pltpu.sync_copy(w_hbm.at[pl.ds(base, B)], w_v)
            pltpu.async_copy(ts_hbm.at[row_v], msg_v, sem).wait()

            def sgroup(kk, c2):
                wvec = w_v[pl.ds(kk * 16, 16)]
                for i in range(16):
                    wi = wvec[i]
                    r = kk * 16 + i
                    for j in range(H // 16):
                        sl = pl.ds(j * 16, 16)
                        msg_v[r, sl] = msg_v[r, sl] * wi
                return c2

            lax.fori_loop(0, B // 16, sgroup, 0)
            pltpu.sync_copy(msg_v, acc_sh.at[col_v], add=True)
            return carry

        lax.fori_loop(0, NBLK, blk, 0)
        plsc.subcore_barrier()
        for kk in range(NCHUNK):
            off = sid * RPT + kk * RCHUNK
            pltpu.sync_copy(acc_sh.at[pl.ds(off, RCHUNK)], msg_v)
            pltpu.sync_copy(msg_v, out_hbm.at[cid, pl.ds(off, RCHUNK)])

    return k(ts, rowp, colp, wp)


BM = 1000


def _dinv(d0_ref, d1_ref):
    deg = d0_ref[:, 0:1] + d1_ref[:, 0:1] + 1.0
    return lax.rsqrt(deg)


def _tc_embed(x2, We, be, W0, d0, d1):
    def body(x_ref, we_ref, be_ref, w0_ref, d0_ref, d1_ref, o_ref):
        dinv = _dinv(d0_ref, d1_ref)
        h = jnp.dot(x_ref[...], we_ref[...],
                    preferred_element_type=jnp.float32) + be_ref[...]
        h = jnp.maximum(h, 0.0)
        t = jnp.dot(h, w0_ref[...], preferred_element_type=jnp.float32)
        o_ref[...] = t * dinv

    return pl.pallas_call(
        body,
        grid=(N // BM,),
        in_specs=[
            pl.BlockSpec((BM, DIN), lambda i: (i, 0)),
            pl.BlockSpec((DIN, H), lambda i: (0, 0)),
            pl.BlockSpec((1, H), lambda i: (0, 0)),
            pl.BlockSpec((H, H), lambda i: (0, 0)),
            pl.BlockSpec((BM, 16), lambda i: (i, 0)),
            pl.BlockSpec((BM, 16), lambda i: (i, 0)),
        ],
        out_specs=pl.BlockSpec((BM, H), lambda i: (i, 0)),
        out_shape=jax.ShapeDtypeStruct((N, H), jnp.float32),
    )(x2, We, be, W0, d0, d1)


def _tc_layer(p0, p1, tsp, d0, d1, b, W):
    def body(p0_ref, p1_ref, tsp_ref, d0_ref, d1_ref, b_ref, w_ref, o_ref):
        dinv = _dinv(d0_ref, d1_ref)
        h = dinv * (p0_ref[...] + p1_ref[...] + tsp_ref[...]) + b_ref[...]
        h = jnp.maximum(h, 0.0)
        t = jnp.dot(h, w_ref[...], preferred_element_type=jnp.float32)
        o_ref[...] = t * dinv

    return pl.pallas_call(
        body,
        grid=(N // BM,),
        in_specs=[
            pl.BlockSpec((BM, H), lambda i: (i, 0)),
            pl.BlockSpec((BM, H), lambda i: (i, 0)),
            pl.BlockSpec((BM, H), lambda i: (i, 0)),
            pl.BlockSpec((BM, 16), lambda i: (i, 0)),
            pl.BlockSpec((BM, 16), lambda i: (i, 0)),
            pl.BlockSpec((1, H), lambda i: (0, 0)),
            pl.BlockSpec((H, H), lambda i: (0, 0)),
        ],
        out_specs=pl.BlockSpec((BM, H), lambda i: (i, 0)),
        out_shape=jax.ShapeDtypeStruct((N, H), jnp.float32),
    )(p0, p1, tsp, d0, d1, b, W)


def _tc_decode(p0, p1, tsp, d0, d1, bg, Wd, bd):
    def body(p0_ref, p1_ref, tsp_ref, d0_ref, d1_ref, bg_ref, wd_ref,
             bd_ref, o_ref):
        dinv = _dinv(d0_ref, d1_ref)
        h = dinv * (p0_ref[...] + p1_ref[...] + tsp_ref[...]) + bg_ref[...]
        h = jnp.maximum(h, 0.0)
        o_ref[...] = jnp.dot(h, wd_ref[...],
                             preferred_element_type=jnp.float32) + bd_ref[...]

    return pl.pallas_call(
        body,
        grid=(N // BM,),
        in_specs=[
            pl.BlockSpec((BM, H), lambda i: (i, 0)),
            pl.BlockSpec((BM, H), lambda i: (i, 0)),
            pl.BlockSpec((BM, H), lambda i: (i, 0)),
            pl.BlockSpec((BM, 16), lambda i: (i, 0)),
            pl.BlockSpec((BM, 16), lambda i: (i, 0)),
            pl.BlockSpec((1, H), lambda i: (0, 0)),
            pl.BlockSpec((H, DOUT), lambda i: (0, 0)),
            pl.BlockSpec((1, DOUT), lambda i: (0, 0)),
        ],
        out_specs=pl.BlockSpec((BM, DOUT), lambda i: (i, 0)),
        out_shape=jax.ShapeDtypeStruct((N, DOUT), jnp.float32),
    )(p0, p1, tsp, d0, d1, bg, Wd, bd)


def kernel(x, edge_index, edge_weights, W_emb, b_emb, W_gcn, b_gcn,
           W_dec, b_dec):
    pad = EPAD - E
    rowp = jnp.concatenate([edge_index[0],
                            jnp.zeros((pad,), edge_index.dtype)])
    colp = jnp.concatenate([edge_index[1],
                            jnp.zeros((pad,), edge_index.dtype)])
    wp = jnp.concatenate([edge_weights, jnp.zeros((pad,), jnp.float32)])

    ones = jnp.ones((N, H), jnp.float32)
    dparts = _msg_scatter(ones, rowp, colp, wp)
    d0, d1 = dparts[0, :, :16], dparts[1, :, :16]

    x2 = x.reshape(N, DIN)
    be = b_emb.reshape(1, H)
    ts = _tc_embed(x2, W_emb, be, W_gcn[0], d0, d1)
    for l in range(1, NLAYERS):
        parts = _msg_scatter(ts, rowp, colp, wp)
        ts = _tc_layer(parts[0], parts[1], ts, d0, d1,
                       b_gcn[l - 1].reshape(1, H), W_gcn[l])
    parts = _msg_scatter(ts, rowp, colp, wp)
    y = _tc_decode(parts[0], parts[1], ts, d0, d1,
                   b_gcn[NLAYERS - 1].reshape(1, H), W_dec,
                   b_dec.reshape(1, DOUT))
    return y.reshape(N, DOUT // 12, 12)
